# speedup vs baseline: 1.4877x; 1.4877x over previous
import jax
import jax.numpy as jnp
from jax.experimental import pallas as pl

SEQ_BLOCK = 512


def _add_kernel(x_ref, pos_ref, out_ref):
    out_ref[...] = x_ref[...] + pos_ref[...][None, :, :]


def kernel(x, pos_table):
    batch, seq_len, dim = x.shape
    sb = SEQ_BLOCK if seq_len % SEQ_BLOCK == 0 else seq_len
    grid = (seq_len // sb, batch)
    return pl.pallas_call(
        _add_kernel,
        grid=grid,
        in_specs=[
            pl.BlockSpec((1, sb, dim), lambda i, j: (j, i, 0)),
            pl.BlockSpec((sb, dim), lambda i, j: (i, 0)),
        ],
        out_specs=pl.BlockSpec((1, sb, dim), lambda i, j: (j, i, 0)),
        out_shape=jax.ShapeDtypeStruct(x.shape, x.dtype),
    )(x, pos_table[:seq_len])


# speedup vs baseline: 1.6605x; 1.1162x over previous
import jax
import jax.numpy as jnp
from jax.experimental import pallas as pl

SEQ_BLOCK = 1024


def _add_kernel(x_ref, pos_ref, out_ref):
    out_ref[...] = x_ref[...] + pos_ref[...][None, :, :]


def kernel(x, pos_table):
    batch, seq_len, dim = x.shape
    sb = SEQ_BLOCK if seq_len % SEQ_BLOCK == 0 else seq_len
    grid = (seq_len // sb, batch)
    return pl.pallas_call(
        _add_kernel,
        grid=grid,
        in_specs=[
            pl.BlockSpec((1, sb, dim), lambda i, j: (j, i, 0)),
            pl.BlockSpec((sb, dim), lambda i, j: (i, 0)),
        ],
        out_specs=pl.BlockSpec((1, sb, dim), lambda i, j: (j, i, 0)),
        out_shape=jax.ShapeDtypeStruct(x.shape, x.dtype),
    )(x, pos_table[:seq_len])


# speedup vs baseline: 1.7378x; 1.0465x over previous
import jax
import jax.numpy as jnp
from jax.experimental import pallas as pl

SEQ_BLOCK = 2048


def _add_kernel(x_ref, pos_ref, out_ref):
    out_ref[...] = x_ref[...] + pos_ref[...][None, :, :]


def kernel(x, pos_table):
    batch, seq_len, dim = x.shape
    sb = SEQ_BLOCK if seq_len % SEQ_BLOCK == 0 else seq_len
    grid = (seq_len // sb, batch)
    return pl.pallas_call(
        _add_kernel,
        grid=grid,
        in_specs=[
            pl.BlockSpec((1, sb, dim), lambda i, j: (j, i, 0)),
            pl.BlockSpec((sb, dim), lambda i, j: (i, 0)),
        ],
        out_specs=pl.BlockSpec((1, sb, dim), lambda i, j: (j, i, 0)),
        out_shape=jax.ShapeDtypeStruct(x.shape, x.dtype),
    )(x, pos_table[:seq_len])
